# baseline (speedup 1.0000x reference)
import functools

import jax
import jax.numpy as jnp
from jax import lax
from jax.experimental import pallas as pl
from jax.experimental.pallas import tpu as pltpu
from jax.experimental.pallas import tpu_sc as plsc

LANES = 128
S0 = 1024
S1 = 1024
NB = 32
N0 = 16384
N1 = 4096
NW = 32
C0 = N0 // NW
C1 = N1 // NW


def _kl(x, y):
    mx = jnp.max(x, axis=0, keepdims=True)
    ex = jnp.exp(x - mx)
    sx = jnp.sum(ex, axis=0, keepdims=True)
    my = jnp.max(y, axis=0, keepdims=True)
    ey = jnp.exp(y - my)
    sy = jnp.sum(ey, axis=0, keepdims=True)
    t = jnp.sum(ex * (x - y), axis=0, keepdims=True) / sx
    return t - (mx + jnp.log(sx)) + (my + jnp.log(sy))


def _scale_body(x_ref, y_ref, out_ref):
    out_ref[0] = _kl(x_ref[0], y_ref[0])


def _kl_call(x, y, s_blk):
    b, c, hw = x.shape
    nb = hw // s_blk
    f = pl.BlockSpec((1, c, s_blk), lambda i: (i // nb, 0, i % nb))
    return pl.pallas_call(
        _scale_body,
        grid=(b * nb,),
        in_specs=[f, f],
        out_specs=pl.BlockSpec((1, 1, s_blk), lambda i: (i // nb, 0, i % nb)),
        out_shape=jax.ShapeDtypeStruct((b, 1, hw), jnp.float32),
    )(x, y)


def _sc_bin_combine(kl0, lab0, kl1, lab1, gate):
    mesh = plsc.VectorSubcoreMesh(core_axis_name="c", subcore_axis_name="s")

    @functools.partial(
        pl.kernel,
        mesh=mesh,
        out_type=jax.ShapeDtypeStruct((16,), jnp.float32),
        compiler_params=pltpu.CompilerParams(needs_layout_passes=False),
        scratch_types=[
            pltpu.VMEM((C0,), jnp.float32),
            pltpu.VMEM((C0,), jnp.int32),
            pltpu.VMEM((C1,), jnp.float32),
            pltpu.VMEM((C1,), jnp.int32),
            pltpu.VMEM((4 * NB * 16,), jnp.float32),
            pltpu.VMEM((4 * NB * 16,), jnp.float32),
            pltpu.VMEM((2 * NB,), jnp.float32),
            pltpu.VMEM((16,), jnp.float32),
            pltpu.VMEM_SHARED((NW, 4 * NB * 16), jnp.float32),
        ],
    )
    def body(kl0_hbm, lab0_hbm, kl1_hbm, lab1_hbm, gate_hbm, out_hbm,
             kl0_v, lab0_v, kl1_v, lab1_v, bins_v, row_v, gate_v, loss_v,
             shared):
        wid = lax.axis_index("s") * 2 + lax.axis_index("c")
        zeros16 = jnp.zeros((16,), jnp.float32)
        ones16 = jnp.ones((16,), jnp.float32)
        lane = lax.iota(jnp.int32, 16)
        for j in range(4 * NB):
            bins_v[pl.ds(j * 16, 16)] = zeros16

        pltpu.sync_copy(kl0_hbm.at[pl.ds(wid * C0, C0)], kl0_v)
        pltpu.sync_copy(lab0_hbm.at[pl.ds(wid * C0, C0)], lab0_v)
        pltpu.sync_copy(kl1_hbm.at[pl.ds(wid * C1, C1)], kl1_v)
        pltpu.sync_copy(lab1_hbm.at[pl.ds(wid * C1, C1)], lab1_v)

        def bin_chunk(n, kl_v, lab_v, base_s, base_c):
            def step(j, carry):
                accs = list(carry)
                labv = lab_v[pl.ds(j * 16, 16)]
                klv = kl_v[pl.ds(j * 16, 16)]
                for cl in range(21):
                    mask = labv == cl
                    accs[cl] = accs[cl] + jnp.where(mask, klv, jnp.float32(0.0))
                    accs[21 + cl] = accs[21 + cl] + jnp.where(mask, ones16, jnp.float32(0.0))
                return tuple(accs)

            init = tuple(jnp.zeros((16,), jnp.float32) for _ in range(42))
            accs = lax.fori_loop(0, n // 16, step, init)
            for cl in range(21):
                bins_v[pl.ds((base_s + cl) * 16, 16)] = accs[cl]
                bins_v[pl.ds((base_c + cl) * 16, 16)] = accs[21 + cl]

        bin_chunk(C0, kl0_v, lab0_v, 0, NB)
        bin_chunk(C1, kl1_v, lab1_v, 2 * NB, 3 * NB)

        pltpu.sync_copy(bins_v, shared.at[wid])
        plsc.subcore_barrier()

        @pl.when(wid == 0)
        def _combine():
            pltpu.sync_copy(gate_hbm, gate_v)

            def add_row(w, _):
                pltpu.sync_copy(shared.at[w], row_v)
                for j in range(4 * NB):
                    sl = pl.ds(j * 16, 16)
                    bins_v[sl] = bins_v[sl] + row_v[sl]
                return _
            lax.fori_loop(1, NW, add_row, 0)

            def totals(qbase):
                lo = jnp.zeros((16,), jnp.float32)
                hi = jnp.zeros((16,), jnp.float32)
                for cl in range(21):
                    s = jnp.sum(bins_v[pl.ds((qbase + cl) * 16, 16)])
                    vec = jnp.full((16,), s, jnp.float32)
                    if cl < 16:
                        lo = jnp.where(lane == cl, vec, lo)
                    else:
                        hi = jnp.where(lane == cl - 16, vec, hi)
                return lo, hi

            def term(sbase, cbase):
                s_lo, s_hi = totals(sbase)
                c_lo, c_hi = totals(cbase)
                t = jnp.float32(0.0)
                for s, c, g_off in ((s_lo, c_lo, 0), (s_hi, c_hi, 16)):
                    g = gate_v[pl.ds(g_off, 16)]
                    klc = s / jnp.maximum(c, jnp.float32(1.0))
                    contrib = g * jnp.where(c > jnp.float32(0.0), klc, jnp.float32(0.0))
                    t = t + jnp.sum(contrib)
                return t

            loss = (term(0, NB) + jnp.float32(2.0) * term(2 * NB, 3 * NB))
            loss_v[...] = jnp.full((16,), loss, jnp.float32)
            pltpu.sync_copy(loss_v, out_hbm)

    return body(kl0, lab0, kl1, lab1, gate)


def kernel(pseudo_labels, feat_old_0, feat_0, feat_old_1, feat_1, num_class, num_old_class):
    b = pseudo_labels.shape[0]

    lab0 = pseudo_labels[:, 0, ::8, ::8].reshape(N0)
    lab1 = pseudo_labels[:, 0, ::16, ::16].reshape(N1)

    x0 = feat_0.reshape(b, 384, 4096)
    y0 = feat_old_0.reshape(b, 384, 4096)
    x1 = feat_1.reshape(b, 768, 1024)
    y1 = feat_old_1.reshape(b, 768, 1024)

    kl0 = _kl_call(x0, y0, S0).reshape(N0)
    kl1 = _kl_call(x1, y1, S1).reshape(N1)

    cls = jnp.arange(2 * NB, dtype=jnp.float32)
    noc = jnp.asarray(num_old_class, jnp.float32)
    nc = jnp.asarray(num_class, jnp.float32)
    gate = jnp.where(
        cls == 0,
        noc / nc,
        jnp.where((cls <= noc) & (cls < 21), jnp.float32(1.0), jnp.float32(0.0)),
    )

    out = _sc_bin_combine(kl0, lab0, kl1, lab1, gate)
    return out[0]


# speedup vs baseline: 1.0749x; 1.0749x over previous
import functools

import jax
import jax.numpy as jnp
from jax import lax
from jax.experimental import pallas as pl
from jax.experimental.pallas import tpu as pltpu
from jax.experimental.pallas import tpu_sc as plsc

LANES = 128
S0 = 1024
S1 = 1024
NB = 32
N0 = 16384
N1 = 4096
NW = 16
C0 = N0 // NW
C1 = N1 // NW
NROW = 4 * NB
COLS = NROW * 16 // NW


def _kl(x, y):
    mx = jnp.max(x, axis=0, keepdims=True)
    ex = jnp.exp(x - mx)
    sx = jnp.sum(ex, axis=0, keepdims=True)
    my = jnp.max(y, axis=0, keepdims=True)
    ey = jnp.exp(y - my)
    sy = jnp.sum(ey, axis=0, keepdims=True)
    t = jnp.sum(ex * (x - y), axis=0, keepdims=True) / sx
    return t - (mx + jnp.log(sx)) + (my + jnp.log(sy))


def _scale_body(x_ref, y_ref, out_ref):
    out_ref[0] = _kl(x_ref[0], y_ref[0])


def _kl_call(x, y, s_blk):
    b, c, hw = x.shape
    nb = hw // s_blk
    f = pl.BlockSpec((1, c, s_blk), lambda i: (i // nb, 0, i % nb))
    return pl.pallas_call(
        _scale_body,
        grid=(b * nb,),
        in_specs=[f, f],
        out_specs=pl.BlockSpec((1, 1, s_blk), lambda i: (i // nb, 0, i % nb)),
        out_shape=jax.ShapeDtypeStruct((b, 1, hw), jnp.float32),
    )(x, y)


def _sc_bin_combine(kl0, lab0, kl1, lab1, gate):
    mesh = plsc.VectorSubcoreMesh(
        core_axis_name="c", subcore_axis_name="s", num_cores=1)

    @functools.partial(
        pl.kernel,
        mesh=mesh,
        out_type=jax.ShapeDtypeStruct((16,), jnp.float32),
        compiler_params=pltpu.CompilerParams(needs_layout_passes=False),
        scratch_types=[
            pltpu.VMEM((C0,), jnp.float32),
            pltpu.VMEM((C0,), jnp.int32),
            pltpu.VMEM((C1,), jnp.float32),
            pltpu.VMEM((C1,), jnp.int32),
            pltpu.VMEM((NROW * 16,), jnp.float32),
            pltpu.VMEM((NW * COLS,), jnp.float32),
            pltpu.VMEM((NROW * 16,), jnp.float32),
            pltpu.VMEM((2 * NB,), jnp.float32),
            pltpu.VMEM((16,), jnp.float32),
            pltpu.SemaphoreType.DMA,
            pltpu.VMEM_SHARED((NW, NROW * 16), jnp.float32),
            pltpu.VMEM_SHARED((NROW * 16,), jnp.float32),
        ],
    )
    def body(kl0_hbm, lab0_hbm, kl1_hbm, lab1_hbm, gate_hbm, out_hbm,
             kl0_v, lab0_v, kl1_v, lab1_v, bins_v, strip_v, tot_v, gate_v,
             loss_v, sem, shared, shared2):
        wid = lax.axis_index("s")
        zeros16 = jnp.zeros((16,), jnp.float32)
        ones16 = jnp.ones((16,), jnp.float32)
        lane = lax.iota(jnp.int32, 16)

        cps = [
            pltpu.async_copy(kl0_hbm.at[pl.ds(wid * C0, C0)], kl0_v, sem),
            pltpu.async_copy(lab0_hbm.at[pl.ds(wid * C0, C0)], lab0_v, sem),
            pltpu.async_copy(kl1_hbm.at[pl.ds(wid * C1, C1)], kl1_v, sem),
            pltpu.async_copy(lab1_hbm.at[pl.ds(wid * C1, C1)], lab1_v, sem),
        ]
        for cp in cps:
            cp.wait()

        def bin_chunk(n, kl_v, lab_v, base_s, base_c):
            def step(j, carry):
                accs = list(carry)
                labv = lab_v[pl.ds(j * 16, 16)]
                klv = kl_v[pl.ds(j * 16, 16)]
                for cl in range(21):
                    mask = labv == cl
                    accs[cl] = accs[cl] + jnp.where(mask, klv, jnp.float32(0.0))
                    accs[21 + cl] = accs[21 + cl] + jnp.where(mask, ones16, jnp.float32(0.0))
                return tuple(accs)

            init = tuple(jnp.zeros((16,), jnp.float32) for _ in range(42))
            accs = lax.fori_loop(0, n // 16, step, init)
            for cl in range(21):
                bins_v[pl.ds((base_s + cl) * 16, 16)] = accs[cl]
                bins_v[pl.ds((base_c + cl) * 16, 16)] = accs[21 + cl]

        for j in range(NROW):
            bins_v[pl.ds(j * 16, 16)] = zeros16
        bin_chunk(C0, kl0_v, lab0_v, 0, NB)
        bin_chunk(C1, kl1_v, lab1_v, 2 * NB, 3 * NB)

        pltpu.sync_copy(bins_v, shared.at[wid])
        plsc.subcore_barrier()

        base = wid * COLS
        for r in range(NW):
            pltpu.sync_copy(shared.at[r, pl.ds(base, COLS)],
                            strip_v.at[pl.ds(r * COLS, COLS)])
        acc = [jnp.zeros((16,), jnp.float32) for _ in range(COLS // 16)]
        for r in range(NW):
            for k in range(COLS // 16):
                acc[k] = acc[k] + strip_v[pl.ds(r * COLS + k * 16, 16)]
        for k in range(COLS // 16):
            strip_v[pl.ds(k * 16, 16)] = acc[k]
        pltpu.sync_copy(strip_v.at[pl.ds(0, COLS)], shared2.at[pl.ds(base, COLS)])
        plsc.subcore_barrier()

        @pl.when(wid == 0)
        def _combine():
            pltpu.sync_copy(gate_hbm, gate_v)
            pltpu.sync_copy(shared2, tot_v)

            def totals(qbase):
                lo = jnp.zeros((16,), jnp.float32)
                hi = jnp.zeros((16,), jnp.float32)
                for cl in range(21):
                    s = jnp.sum(tot_v[pl.ds((qbase + cl) * 16, 16)])
                    vec = jnp.full((16,), s, jnp.float32)
                    if cl < 16:
                        lo = jnp.where(lane == cl, vec, lo)
                    else:
                        hi = jnp.where(lane == cl - 16, vec, hi)
                return lo, hi

            def term(sbase, cbase):
                s_lo, s_hi = totals(sbase)
                c_lo, c_hi = totals(cbase)
                t = jnp.float32(0.0)
                for s, c, g_off in ((s_lo, c_lo, 0), (s_hi, c_hi, 16)):
                    g = gate_v[pl.ds(g_off, 16)]
                    klc = s / jnp.maximum(c, jnp.float32(1.0))
                    contrib = g * jnp.where(c > jnp.float32(0.0), klc, jnp.float32(0.0))
                    t = t + jnp.sum(contrib)
                return t

            loss = (term(0, NB) + jnp.float32(2.0) * term(2 * NB, 3 * NB))
            loss_v[...] = jnp.full((16,), loss, jnp.float32)
            pltpu.sync_copy(loss_v, out_hbm)

    return body(kl0, lab0, kl1, lab1, gate)


def kernel(pseudo_labels, feat_old_0, feat_0, feat_old_1, feat_1, num_class, num_old_class):
    b = pseudo_labels.shape[0]

    lab0 = pseudo_labels[:, 0, ::8, ::8].reshape(N0)
    lab1 = pseudo_labels[:, 0, ::16, ::16].reshape(N1)

    x0 = feat_0.reshape(b, 384, 4096)
    y0 = feat_old_0.reshape(b, 384, 4096)
    x1 = feat_1.reshape(b, 768, 1024)
    y1 = feat_old_1.reshape(b, 768, 1024)

    kl0 = _kl_call(x0, y0, S0).reshape(N0)
    kl1 = _kl_call(x1, y1, S1).reshape(N1)

    cls = jnp.arange(2 * NB, dtype=jnp.float32)
    noc = jnp.asarray(num_old_class, jnp.float32)
    nc = jnp.asarray(num_class, jnp.float32)
    gate = jnp.where(
        cls == 0,
        noc / nc,
        jnp.where((cls <= noc) & (cls < 21), jnp.float32(1.0), jnp.float32(0.0)),
    )

    out = _sc_bin_combine(kl0, lab0, kl1, lab1, gate)
    return out[0]


# speedup vs baseline: 1.0833x; 1.0078x over previous
import functools

import jax
import jax.numpy as jnp
from jax import lax
from jax.experimental import pallas as pl
from jax.experimental.pallas import tpu as pltpu
from jax.experimental.pallas import tpu_sc as plsc

LANES = 128
S0 = 1024
S1 = 1024
NB = 32
N0 = 16384
N1 = 4096
NW = 16
NROW = 2 * NB
COLS = NROW * 16 // NW


def _kl(x, y):
    mx = jnp.max(x, axis=0, keepdims=True)
    ex = jnp.exp(x - mx)
    sx = jnp.sum(ex, axis=0, keepdims=True)
    my = jnp.max(y, axis=0, keepdims=True)
    ey = jnp.exp(y - my)
    sy = jnp.sum(ey, axis=0, keepdims=True)
    t = jnp.sum(ex * (x - y), axis=0, keepdims=True) / sx
    return t - (mx + jnp.log(sx)) + (my + jnp.log(sy))


def _scale_body(x_ref, y_ref, out_ref):
    out_ref[0] = _kl(x_ref[0], y_ref[0])


def _kl_call(x, y, s_blk):
    b, c, hw = x.shape
    nb = hw // s_blk
    f = pl.BlockSpec((1, c, s_blk), lambda i: (i // nb, 0, i % nb))
    return pl.pallas_call(
        _scale_body,
        grid=(b * nb,),
        in_specs=[f, f],
        out_specs=pl.BlockSpec((1, 1, s_blk), lambda i: (i // nb, 0, i % nb)),
        out_shape=jax.ShapeDtypeStruct((b, 1, hw), jnp.float32),
    )(x, y)


def _mesh():
    return plsc.VectorSubcoreMesh(
        core_axis_name="c", subcore_axis_name="s", num_cores=1)


def _bin_and_reduce(n_pix, kl_hbm, lab_hbm, kl_v, lab_v, bins_v, strip_v,
                    sem, shared, wid):
    cpw = n_pix // NW
    ones16 = jnp.ones((16,), jnp.float32)

    cps = [
        pltpu.async_copy(kl_hbm.at[pl.ds(wid * cpw, cpw)], kl_v, sem),
        pltpu.async_copy(lab_hbm.at[pl.ds(wid * cpw, cpw)], lab_v, sem),
    ]
    for cp in cps:
        cp.wait()

    def step(j, carry):
        accs = list(carry)
        labv = lab_v[pl.ds(j * 16, 16)]
        klv = kl_v[pl.ds(j * 16, 16)]
        for cl in range(21):
            mask = labv == cl
            accs[cl] = accs[cl] + jnp.where(mask, klv, jnp.float32(0.0))
            accs[21 + cl] = accs[21 + cl] + jnp.where(mask, ones16, jnp.float32(0.0))
        return tuple(accs)

    init = tuple(jnp.zeros((16,), jnp.float32) for _ in range(42))
    accs = lax.fori_loop(0, cpw // 16, step, init)
    zeros16 = jnp.zeros((16,), jnp.float32)
    for j in range(NROW):
        bins_v[pl.ds(j * 16, 16)] = zeros16
    for cl in range(21):
        bins_v[pl.ds(cl * 16, 16)] = accs[cl]
        bins_v[pl.ds((NB + cl) * 16, 16)] = accs[21 + cl]

    pltpu.sync_copy(bins_v, shared.at[wid])
    plsc.subcore_barrier()

    base = wid * COLS
    for r in range(NW):
        pltpu.sync_copy(shared.at[r, pl.ds(base, COLS)],
                        strip_v.at[pl.ds(r * COLS, COLS)])
    acc = [jnp.zeros((16,), jnp.float32) for _ in range(COLS // 16)]
    for r in range(NW):
        for k in range(COLS // 16):
            acc[k] = acc[k] + strip_v[pl.ds(r * COLS + k * 16, 16)]
    for k in range(COLS // 16):
        strip_v[pl.ds(k * 16, 16)] = acc[k]


def _sc_bin0(kl0, lab0):
    cpw = N0 // NW

    @functools.partial(
        pl.kernel,
        mesh=_mesh(),
        out_type=jax.ShapeDtypeStruct((NROW * 16,), jnp.float32),
        compiler_params=pltpu.CompilerParams(needs_layout_passes=False),
        scratch_types=[
            pltpu.VMEM((cpw,), jnp.float32),
            pltpu.VMEM((cpw,), jnp.int32),
            pltpu.VMEM((NROW * 16,), jnp.float32),
            pltpu.VMEM((NW * COLS,), jnp.float32),
            pltpu.SemaphoreType.DMA,
            pltpu.VMEM_SHARED((NW, NROW * 16), jnp.float32),
        ],
    )
    def body(kl_hbm, lab_hbm, out_hbm, kl_v, lab_v, bins_v, strip_v, sem,
             shared):
        wid = lax.axis_index("s")
        _bin_and_reduce(N0, kl_hbm, lab_hbm, kl_v, lab_v, bins_v, strip_v,
                        sem, shared, wid)
        pltpu.sync_copy(strip_v.at[pl.ds(0, COLS)],
                        out_hbm.at[pl.ds(wid * COLS, COLS)])

    return body(kl0, lab0)


def _sc_bin1_combine(kl1, lab1, tot0, gate):
    cpw = N1 // NW

    @functools.partial(
        pl.kernel,
        mesh=_mesh(),
        out_type=jax.ShapeDtypeStruct((16,), jnp.float32),
        compiler_params=pltpu.CompilerParams(needs_layout_passes=False),
        scratch_types=[
            pltpu.VMEM((cpw,), jnp.float32),
            pltpu.VMEM((cpw,), jnp.int32),
            pltpu.VMEM((NROW * 16,), jnp.float32),
            pltpu.VMEM((NW * COLS,), jnp.float32),
            pltpu.VMEM((NROW * 16,), jnp.float32),
            pltpu.VMEM((NROW * 16,), jnp.float32),
            pltpu.VMEM((2 * NB,), jnp.float32),
            pltpu.VMEM((16,), jnp.float32),
            pltpu.SemaphoreType.DMA,
            pltpu.VMEM_SHARED((NW, NROW * 16), jnp.float32),
            pltpu.VMEM_SHARED((NROW * 16,), jnp.float32),
        ],
    )
    def body(kl_hbm, lab_hbm, tot0_hbm, gate_hbm, out_hbm, kl_v, lab_v,
             bins_v, strip_v, tot0_v, tot1_v, gate_v, loss_v, sem, shared,
             shared2):
        wid = lax.axis_index("s")
        lane = lax.iota(jnp.int32, 16)
        _bin_and_reduce(N1, kl_hbm, lab_hbm, kl_v, lab_v, bins_v, strip_v,
                        sem, shared, wid)
        pltpu.sync_copy(strip_v.at[pl.ds(0, COLS)],
                        shared2.at[pl.ds(wid * COLS, COLS)])
        plsc.subcore_barrier()

        @pl.when(wid == 0)
        def _combine():
            pltpu.sync_copy(gate_hbm, gate_v)
            pltpu.sync_copy(tot0_hbm, tot0_v)
            pltpu.sync_copy(shared2, tot1_v)

            def totals(tv, qbase):
                lo = jnp.zeros((16,), jnp.float32)
                hi = jnp.zeros((16,), jnp.float32)
                for cl in range(21):
                    s = jnp.sum(tv[pl.ds((qbase + cl) * 16, 16)])
                    vec = jnp.full((16,), s, jnp.float32)
                    if cl < 16:
                        lo = jnp.where(lane == cl, vec, lo)
                    else:
                        hi = jnp.where(lane == cl - 16, vec, hi)
                return lo, hi

            def term(tv):
                s_lo, s_hi = totals(tv, 0)
                c_lo, c_hi = totals(tv, NB)
                t = jnp.float32(0.0)
                for s, c, g_off in ((s_lo, c_lo, 0), (s_hi, c_hi, 16)):
                    g = gate_v[pl.ds(g_off, 16)]
                    klc = s / jnp.maximum(c, jnp.float32(1.0))
                    contrib = g * jnp.where(c > jnp.float32(0.0), klc, jnp.float32(0.0))
                    t = t + jnp.sum(contrib)
                return t

            loss = term(tot0_v) + jnp.float32(2.0) * term(tot1_v)
            loss_v[...] = jnp.full((16,), loss, jnp.float32)
            pltpu.sync_copy(loss_v, out_hbm)

    return body(kl1, lab1, tot0, gate)


def kernel(pseudo_labels, feat_old_0, feat_0, feat_old_1, feat_1, num_class, num_old_class):
    b = pseudo_labels.shape[0]

    lab0 = pseudo_labels[:, 0, ::8, ::8].reshape(N0)
    lab1 = pseudo_labels[:, 0, ::16, ::16].reshape(N1)

    x0 = feat_0.reshape(b, 384, 4096)
    y0 = feat_old_0.reshape(b, 384, 4096)
    x1 = feat_1.reshape(b, 768, 1024)
    y1 = feat_old_1.reshape(b, 768, 1024)

    kl0 = _kl_call(x0, y0, S0).reshape(N0)
    tot0 = _sc_bin0(kl0, lab0)
    kl1 = _kl_call(x1, y1, S1).reshape(N1)

    cls = jnp.arange(2 * NB, dtype=jnp.float32)
    noc = jnp.asarray(num_old_class, jnp.float32)
    nc = jnp.asarray(num_class, jnp.float32)
    gate = jnp.where(
        cls == 0,
        noc / nc,
        jnp.where((cls <= noc) & (cls < 21), jnp.float32(1.0), jnp.float32(0.0)),
    )

    out = _sc_bin1_combine(kl1, lab1, tot0, gate)
    return out[0]
